# baseline (speedup 1.0000x reference)
import jax
import jax.numpy as jnp
from jax.experimental import pallas as pl
from jax.experimental.pallas import tpu as pltpu


def _gat_body(inc_ref, ef_ref, nf_ref, Wns_ref, bns_ref, Wes_ref, bes_ref,
              War_ref, bar_ref, Wob_ref, boc_ref,
              updo_ref, mnp_ref, mxp_ref):
    te = (jnp.dot(ef_ref[...], Wes_ref[...],
                  preferred_element_type=jnp.float32)
          + bes_ref[0:1, :]).astype(jnp.bfloat16)

    inc = inc_ref[...]
    acc = jnp.dot(inc.astype(jnp.bfloat16), te,
                  preferred_element_type=jnp.float32)
    rs = jnp.sum(inc, axis=1, keepdims=True)
    agg = acc / (rs + 1e-8)

    tn = jnp.dot(nf_ref[...], Wns_ref[...],
                 preferred_element_type=jnp.float32) + bns_ref[0:1, :]
    att = tn + agg
    sc = jnp.dot(att.astype(jnp.bfloat16), War_ref[...],
                 preferred_element_type=jnp.float32) + bar_ref[0:1, :]
    sc = jnp.where(sc >= 0, sc, 0.2 * sc)
    coeff = jax.nn.sigmoid(sc)
    upd = coeff * agg + tn
    updo = jnp.dot(upd.astype(jnp.bfloat16), Wob_ref[...],
                   preferred_element_type=jnp.float32) + boc_ref[0:1, :]
    updo_ref[...] = updo.astype(jnp.bfloat16)

    mnp_ref[...] = jnp.broadcast_to(
        jnp.min(updo, axis=0, keepdims=True), mnp_ref.shape)
    mxp_ref[...] = jnp.broadcast_to(
        jnp.max(updo, axis=0, keepdims=True), mxp_ref.shape)


def _final_body(updo_ref, mnp_ref, mxp_ref, Wt_ref, bt_ref, out_ref):
    mn = jnp.min(mnp_ref[...], axis=0, keepdims=True)
    mx = jnp.max(mxp_ref[...], axis=0, keepdims=True)
    scale = 1.0 / (mx - mn + 1e-8)
    normed = (updo_ref[...].astype(jnp.float32) - mn) * scale
    out_ref[...] = jnp.dot(normed.astype(jnp.bfloat16), Wt_ref[...],
                           preferred_element_type=jnp.float32) + bt_ref[0:1, :]


def kernel(node_features, incidence_matrix, edge_features,
           Wn, bn, We, be, Wa, ba, Wo, bo, Wt, bt):
    N, NODE_DIM = node_features.shape
    E = incidence_matrix.shape[1]
    EDGE_DIM = edge_features.shape[1]
    H, _, HID = Wn.shape
    OUT = Wo.shape[2]
    HH = H * HID
    HO = H * OUT

    BN = 400
    ni = N // BN

    f32 = jnp.float32
    bf16 = jnp.bfloat16

    Wn_s = Wn.transpose(1, 0, 2).reshape(NODE_DIM, HH).astype(bf16)
    bn_s = jnp.broadcast_to(bn.reshape(1, HH), (8, HH))
    We_s = We.transpose(1, 0, 2).reshape(EDGE_DIM, HH)
    be_s = jnp.broadcast_to(be.reshape(1, HH), (8, HH))
    Wa_rep = jax.scipy.linalg.block_diag(
        *[jnp.tile(Wa[h], (1, HID)) for h in range(H)]).astype(bf16)
    ba_rep = jnp.broadcast_to(
        jnp.repeat(ba.reshape(H, 1), HID, axis=1).reshape(1, HH), (8, HH))
    Wo_bd = jax.scipy.linalg.block_diag(
        *[Wo[h] for h in range(H)]).astype(bf16)
    bo_c = jnp.broadcast_to(bo.reshape(1, HO), (8, HO))
    bt_b = jnp.broadcast_to(bt.reshape(1, OUT), (8, OUT))
    Wt_bf = Wt.astype(bf16)
    nf_bf = node_features.astype(bf16)

    full = lambda shape: pl.BlockSpec(shape, lambda i: (0, 0))
    par = pltpu.CompilerParams(dimension_semantics=("parallel",))

    updo, mnp, mxp = pl.pallas_call(
        _gat_body,
        grid=(ni,),
        in_specs=[
            pl.BlockSpec((BN, E), lambda i: (i, 0)),
            full((E, EDGE_DIM)),
            pl.BlockSpec((BN, NODE_DIM), lambda i: (i, 0)),
            full((NODE_DIM, HH)), full((8, HH)),
            full((EDGE_DIM, HH)), full((8, HH)),
            full((HH, HH)), full((8, HH)),
            full((HH, HO)), full((8, HO)),
        ],
        out_specs=[
            pl.BlockSpec((BN, HO), lambda i: (i, 0)),
            pl.BlockSpec((8, HO), lambda i: (i, 0)),
            pl.BlockSpec((8, HO), lambda i: (i, 0)),
        ],
        out_shape=[
            jax.ShapeDtypeStruct((N, HO), bf16),
            jax.ShapeDtypeStruct((ni * 8, HO), f32),
            jax.ShapeDtypeStruct((ni * 8, HO), f32),
        ],
        compiler_params=par,
    )(incidence_matrix, edge_features, nf_bf,
      Wn_s, bn_s, We_s, be_s, Wa_rep, ba_rep, Wo_bd, bo_c)

    out = pl.pallas_call(
        _final_body,
        grid=(ni,),
        in_specs=[
            pl.BlockSpec((BN, HO), lambda i: (i, 0)),
            full((ni * 8, HO)),
            full((ni * 8, HO)),
            full((HO, OUT)),
            full((8, OUT)),
        ],
        out_specs=pl.BlockSpec((BN, OUT), lambda i: (i, 0)),
        out_shape=jax.ShapeDtypeStruct((N, OUT), f32),
        compiler_params=par,
    )(updo, mnp, mxp, Wt_bf, bt_b)

    return out
